# Initial kernel scaffold; baseline (speedup 1.0000x reference)
import jax
import jax.numpy as jnp
from jax.experimental import pallas as pl


def kernel(lit_feat, clause_feat, edge_lit, edge_clause, W_l2c_0, b_l2c_0, W_c2l_0, b_c2l_0, W_l2c_1, b_l2c_1, W_c2l_1, b_c2l_1):
    raise NotImplementedError("write your pallas kernel here")


# speedup vs baseline: 2.8447x; 2.8447x over previous
import functools

import jax
import jax.numpy as jnp
from jax import lax
from jax.experimental import pallas as pl
from jax.experimental.pallas import tpu as pltpu
from jax.experimental.pallas import tpu_sc as plsc

N_LIT = 50000
N_CLAUSE = 150000
N_EDGES = 800000

_NC = 2
_NS = 16
_KE = 1000
_EPC = N_EDGES // _NS
_NCH = _EPC // _KE
_RC = 1000


def _segsum_sc(nseg):
    nrch = nseg // _RC
    nz = -(-nrch // _NS)
    mesh = plsc.VectorSubcoreMesh(core_axis_name="c", subcore_axis_name="s")

    @functools.partial(
        pl.kernel, mesh=mesh,
        compiler_params=pltpu.CompilerParams(use_tc_tiling_on_sc=False),
        out_type=jax.ShapeDtypeStruct((8 * nseg, 8), jnp.float32),
        scratch_types=[
            pltpu.VMEM((_KE,), jnp.int32),
            pltpu.VMEM((_KE,), jnp.int32),
            pltpu.VMEM((_KE, 8), jnp.float32),
            pltpu.VMEM((_RC, 8), jnp.float32),
            pltpu.VMEM_SHARED((nseg, 8), jnp.float32),
            pltpu.SemaphoreType.DMA,
        ],
    )
    def k(tbl, gidx, sidx, zeros, out, gv, sv, mv, stg, acc, sem):
        cid = lax.axis_index("c")
        sid = lax.axis_index("s")
        for p in range(4):
            sl = cid * 4 + p

            def zbody(z, _):
                j = sid + z * _NS

                @pl.when(j < nrch)
                def _():
                    pltpu.sync_copy(zeros, acc.at[pl.ds(j * _RC, _RC)])
                return _
            lax.fori_loop(0, nz, zbody, None)
            plsc.subcore_barrier()

            def ebody(i, _):
                base = sid * _EPC + i * _KE
                pltpu.sync_copy(gidx.at[pl.ds(sl * N_EDGES + base, _KE)], gv)
                pltpu.sync_copy(sidx.at[pl.ds(base, _KE)], sv)
                pltpu.async_copy(tbl.at[gv], mv, sem).wait()
                pltpu.sync_copy(mv, acc.at[sv], add=True)
                return _
            lax.fori_loop(0, _NCH, ebody, None)
            plsc.subcore_barrier()

            def dbody(z, _):
                j = sid + z * _NS

                @pl.when(j < nrch)
                def _():
                    pltpu.sync_copy(acc.at[pl.ds(j * _RC, _RC)], stg)
                    pltpu.sync_copy(
                        stg, out.at[pl.ds(sl * nseg + j * _RC, _RC)])
                return _
            lax.fori_loop(0, nz, dbody, None)
            plsc.subcore_barrier()

    return k


def _segcount_sc(nseg):
    nrch = nseg // _RC
    nz = -(-nrch // _NS)
    mesh = plsc.VectorSubcoreMesh(core_axis_name="c", subcore_axis_name="s")

    @functools.partial(
        pl.kernel, mesh=mesh,
        compiler_params=pltpu.CompilerParams(use_tc_tiling_on_sc=False),
        out_type=jax.ShapeDtypeStruct((nseg, 8), jnp.float32),
        scratch_types=[
            pltpu.VMEM((_KE,), jnp.int32),
            pltpu.VMEM((_KE, 8), jnp.float32),
            pltpu.VMEM((_RC, 8), jnp.float32),
            pltpu.VMEM_SHARED((nseg, 8), jnp.float32),
        ],
    )
    def k(sidx, ones, zeros, out, sv, mv, stg, acc):
        cid = lax.axis_index("c")
        sid = lax.axis_index("s")

        @pl.when(cid == 0)
        def _():
            pltpu.sync_copy(ones, mv)

            def zbody(z, _):
                j = sid + z * _NS

                @pl.when(j < nrch)
                def _():
                    pltpu.sync_copy(zeros, acc.at[pl.ds(j * _RC, _RC)])
                return _
            lax.fori_loop(0, nz, zbody, None)
            plsc.subcore_barrier()

            def ebody(i, _):
                base = sid * _EPC + i * _KE
                pltpu.sync_copy(sidx.at[pl.ds(base, _KE)], sv)
                pltpu.sync_copy(mv, acc.at[sv], add=True)
                return _
            lax.fori_loop(0, _NCH, ebody, None)
            plsc.subcore_barrier()

            def dbody(z, _):
                j = sid + z * _NS

                @pl.when(j < nrch)
                def _():
                    pltpu.sync_copy(acc.at[pl.ds(j * _RC, _RC)], stg)
                    pltpu.sync_copy(stg, out.at[pl.ds(j * _RC, _RC)])
                return _
            lax.fori_loop(0, nz, dbody, None)

    return k


_BLK = 1000


def _tc_linear(x, w, b):
    n, kdim = x.shape

    def body(x_ref, w_ref, b_ref, o_ref):
        o_ref[...] = (
            jnp.dot(x_ref[...], w_ref[...],
                    preferred_element_type=jnp.float32) + b_ref[...])

    return pl.pallas_call(
        body,
        grid=(n // _BLK,),
        in_specs=[
            pl.BlockSpec((_BLK, kdim), lambda i: (i, 0)),
            pl.BlockSpec((kdim, 64), lambda i: (0, 0)),
            pl.BlockSpec((1, 64), lambda i: (0, 0)),
        ],
        out_specs=pl.BlockSpec((_BLK, 64), lambda i: (i, 0)),
        out_shape=jax.ShapeDtypeStruct((n, 64), jnp.float32),
    )(x, w, b.reshape(1, 64))


def _tc_c2l(sums, cnt, clause_x, w, b):
    n = sums.shape[0]

    def body(s_ref, c_ref, z_ref, w_ref, b_ref, o_ref):
        cnt_col = jnp.maximum(c_ref[...][:, 0:1], 1.0)
        h = jnp.maximum(s_ref[...] / cnt_col, 0.0)
        o_ref[...] = (
            jnp.dot(h, w_ref[0:64, :], preferred_element_type=jnp.float32)
            + z_ref[...] * w_ref[64:65, :] + b_ref[...])

    return pl.pallas_call(
        body,
        grid=(n // _BLK,),
        in_specs=[
            pl.BlockSpec((_BLK, 64), lambda i: (i, 0)),
            pl.BlockSpec((_BLK, 8), lambda i: (i, 0)),
            pl.BlockSpec((_BLK, 1), lambda i: (i, 0)),
            pl.BlockSpec((65, 64), lambda i: (0, 0)),
            pl.BlockSpec((1, 64), lambda i: (0, 0)),
        ],
        out_specs=pl.BlockSpec((_BLK, 64), lambda i: (i, 0)),
        out_shape=jax.ShapeDtypeStruct((n, 64), jnp.float32),
    )(sums, cnt, clause_x, w, b.reshape(1, 64))


def _tc_mean_relu(sums, cnt):
    n = sums.shape[0]

    def body(s_ref, c_ref, o_ref):
        cnt_col = jnp.maximum(c_ref[...][:, 0:1], 1.0)
        o_ref[...] = jnp.maximum(s_ref[...] / cnt_col, 0.0)

    return pl.pallas_call(
        body,
        grid=(n // _BLK,),
        in_specs=[
            pl.BlockSpec((_BLK, 64), lambda i: (i, 0)),
            pl.BlockSpec((_BLK, 8), lambda i: (i, 0)),
        ],
        out_specs=pl.BlockSpec((_BLK, 64), lambda i: (i, 0)),
        out_shape=jax.ShapeDtypeStruct((n, 64), jnp.float32),
    )(sums, cnt)


def _to_sliced(x):
    n = x.shape[0]
    return x.reshape(n, 8, 8).transpose(1, 0, 2).reshape(8 * n, 8)


def _from_sliced(y, n):
    return y.reshape(8, n, 8).transpose(1, 0, 2).reshape(n, 64)


def _tie(embs):
    n, v = embs.shape[0] // 2, embs.shape[1]
    y = embs.reshape(n, 2, v)
    pos, neg = y[:, 0, :], y[:, 1, :]
    cp = jnp.concatenate([pos, neg], axis=1)
    cn = jnp.concatenate([neg, pos], axis=1)
    return jnp.stack((cp, cn), axis=1).reshape(2 * n, 2 * v)


def kernel(lit_feat, clause_feat, edge_lit, edge_clause,
           W_l2c_0, b_l2c_0, W_c2l_0, b_c2l_0,
           W_l2c_1, b_l2c_1, W_c2l_1, b_c2l_1):
    el = jnp.asarray(edge_lit, jnp.int32)
    ec = jnp.asarray(edge_clause, jnp.int32)
    offs_l = (jnp.arange(8, dtype=jnp.int32) * N_LIT)[:, None]
    offs_c = (jnp.arange(8, dtype=jnp.int32) * N_CLAUSE)[:, None]
    gidx_l = (el[None, :] + offs_l).reshape(-1)
    gidx_c = (ec[None, :] + offs_c).reshape(-1)
    zeros = jnp.zeros((_RC, 8), jnp.float32)
    ones = jnp.ones((_KE, 8), jnp.float32)

    seg_c = _segsum_sc(N_CLAUSE)
    seg_l = _segsum_sc(N_LIT)
    cnt_c = _segcount_sc(N_CLAUSE)(ec, ones, zeros)
    cnt_l = _segcount_sc(N_LIT)(el, ones, zeros)

    def layer(lit_x, wl, bl, wc, bc):
        wh = _tc_linear(lit_x, wl, bl)
        s_c = seg_c(_to_sliced(wh), gidx_l, ec, zeros)
        wh2 = _tc_c2l(_from_sliced(s_c, N_CLAUSE), cnt_c, clause_feat, wc, bc)
        s_l = seg_l(_to_sliced(wh2), gidx_c, el, zeros)
        return _tc_mean_relu(_from_sliced(s_l, N_LIT), cnt_l)

    embs = _tie(layer(lit_feat, W_l2c_0, b_l2c_0, W_c2l_0, b_c2l_0))
    pre = layer(embs, W_l2c_1, b_l2c_1, W_c2l_1, b_c2l_1)
    return _tie(pre)


# speedup vs baseline: 3.5735x; 1.2562x over previous
import functools

import jax
import jax.numpy as jnp
from jax import lax
from jax.experimental import pallas as pl
from jax.experimental.pallas import tpu as pltpu
from jax.experimental.pallas import tpu_sc as plsc

N_LIT = 50000
N_CLAUSE = 150000
N_EDGES = 800000

_NC = 2
_NS = 16
_KE = 1000
_KEG = 2000
_EPC = N_EDGES // _NS
_NCH = _EPC // _KE
_NCHG = _EPC // _KEG
_RC = 1000


def _segsum_sc(nseg):
    nrch = nseg // _RC
    nz = -(-nrch // _NS)
    mesh = plsc.VectorSubcoreMesh(core_axis_name="c", subcore_axis_name="s")

    @functools.partial(
        pl.kernel, mesh=mesh,
        compiler_params=pltpu.CompilerParams(use_tc_tiling_on_sc=False),
        out_type=jax.ShapeDtypeStruct((8 * nseg, 8), jnp.float32),
        scratch_types=[
            pltpu.VMEM((_KEG,), jnp.int32),
            pltpu.VMEM((_KEG,), jnp.int32),
            pltpu.VMEM((_KEG,), jnp.int32),
            pltpu.VMEM((_KEG,), jnp.int32),
            pltpu.VMEM((_KEG, 8), jnp.float32),
            pltpu.VMEM((_KEG, 8), jnp.float32),
            pltpu.VMEM((_RC, 8), jnp.float32),
            pltpu.VMEM_SHARED((nseg, 8), jnp.float32),
            pltpu.SemaphoreType.DMA,
            pltpu.SemaphoreType.DMA,
        ],
    )
    def k(tbl, gidx, sidx, zeros, out,
          gv0, gv1, sv0, sv1, mv0, mv1, stg, acc, sem0, sem1):
        cid = lax.axis_index("c")
        sid = lax.axis_index("s")
        gvs, svs, mvs, sems = (gv0, gv1), (sv0, sv1), (mv0, mv1), (sem0, sem1)
        for p in range(4):
            sl = cid * 4 + p

            def zbody(z, _):
                j = sid + z * _NS

                @pl.when(j < nrch)
                def _():
                    pltpu.sync_copy(zeros, acc.at[pl.ds(j * _RC, _RC)])
                return _
            lax.fori_loop(0, nz, zbody, None)
            plsc.subcore_barrier()

            def issue(i, b):
                base = sid * _EPC + i * _KEG
                pltpu.sync_copy(
                    gidx.at[pl.ds(sl * N_EDGES + base, _KEG)], gvs[b])
                pltpu.sync_copy(sidx.at[pl.ds(base, _KEG)], svs[b])
                pltpu.async_copy(tbl.at[gvs[b]], mvs[b], sems[b])

            issue(0, 0)

            def ebody(i2, _):
                for b in range(2):
                    i = i2 * 2 + b

                    @pl.when(i + 1 < _NCHG)
                    def _():
                        issue(i + 1, (b + 1) % 2)
                    pltpu.make_async_copy(
                        tbl.at[gvs[b]], mvs[b], sems[b]).wait()
                    pltpu.sync_copy(mvs[b], acc.at[svs[b]], add=True)
                return _
            lax.fori_loop(0, _NCHG // 2, ebody, None)
            if _NCHG % 2:
                b = (_NCHG - 1) % 2
                pltpu.make_async_copy(tbl.at[gvs[b]], mvs[b], sems[b]).wait()
                pltpu.sync_copy(mvs[b], acc.at[svs[b]], add=True)
            plsc.subcore_barrier()

            def dbody(z, _):
                j = sid + z * _NS

                @pl.when(j < nrch)
                def _():
                    pltpu.sync_copy(acc.at[pl.ds(j * _RC, _RC)], stg)
                    pltpu.sync_copy(
                        stg, out.at[pl.ds(sl * nseg + j * _RC, _RC)])
                return _
            lax.fori_loop(0, nz, dbody, None)
            plsc.subcore_barrier()

    return k


def _segcount_sc(nseg):
    nrch = nseg // _RC
    nz = -(-nrch // _NS)
    mesh = plsc.VectorSubcoreMesh(core_axis_name="c", subcore_axis_name="s")

    @functools.partial(
        pl.kernel, mesh=mesh,
        compiler_params=pltpu.CompilerParams(use_tc_tiling_on_sc=False),
        out_type=jax.ShapeDtypeStruct((nseg, 8), jnp.float32),
        scratch_types=[
            pltpu.VMEM((_KE,), jnp.int32),
            pltpu.VMEM((_KE, 8), jnp.float32),
            pltpu.VMEM((_RC, 8), jnp.float32),
            pltpu.VMEM_SHARED((nseg, 8), jnp.float32),
        ],
    )
    def k(sidx, ones, zeros, out, sv, mv, stg, acc):
        cid = lax.axis_index("c")
        sid = lax.axis_index("s")

        @pl.when(cid == 0)
        def _():
            pltpu.sync_copy(ones, mv)

            def zbody(z, _):
                j = sid + z * _NS

                @pl.when(j < nrch)
                def _():
                    pltpu.sync_copy(zeros, acc.at[pl.ds(j * _RC, _RC)])
                return _
            lax.fori_loop(0, nz, zbody, None)
            plsc.subcore_barrier()

            def ebody(i, _):
                base = sid * _EPC + i * _KE
                pltpu.sync_copy(sidx.at[pl.ds(base, _KE)], sv)
                pltpu.sync_copy(mv, acc.at[sv], add=True)
                return _
            lax.fori_loop(0, _NCH, ebody, None)
            plsc.subcore_barrier()

            def dbody(z, _):
                j = sid + z * _NS

                @pl.when(j < nrch)
                def _():
                    pltpu.sync_copy(acc.at[pl.ds(j * _RC, _RC)], stg)
                    pltpu.sync_copy(stg, out.at[pl.ds(j * _RC, _RC)])
                return _
            lax.fori_loop(0, nz, dbody, None)

    return k


_BLK = 1000


def _tc_linear(x, w, b):
    n, kdim = x.shape

    def body(x_ref, w_ref, b_ref, o_ref):
        o_ref[...] = (
            jnp.dot(x_ref[...], w_ref[...],
                    preferred_element_type=jnp.float32) + b_ref[...])

    return pl.pallas_call(
        body,
        grid=(n // _BLK,),
        in_specs=[
            pl.BlockSpec((_BLK, kdim), lambda i: (i, 0)),
            pl.BlockSpec((kdim, 64), lambda i: (0, 0)),
            pl.BlockSpec((1, 64), lambda i: (0, 0)),
        ],
        out_specs=pl.BlockSpec((_BLK, 64), lambda i: (i, 0)),
        out_shape=jax.ShapeDtypeStruct((n, 64), jnp.float32),
    )(x, w, b.reshape(1, 64))


def _tc_c2l(sums, cnt, clause_x, w, b):
    n = sums.shape[0]

    def body(s_ref, c_ref, z_ref, w_ref, b_ref, o_ref):
        cnt_col = jnp.maximum(c_ref[...][:, 0:1], 1.0)
        h = jnp.maximum(s_ref[...] / cnt_col, 0.0)
        o_ref[...] = (
            jnp.dot(h, w_ref[0:64, :], preferred_element_type=jnp.float32)
            + z_ref[...] * w_ref[64:65, :] + b_ref[...])

    return pl.pallas_call(
        body,
        grid=(n // _BLK,),
        in_specs=[
            pl.BlockSpec((_BLK, 64), lambda i: (i, 0)),
            pl.BlockSpec((_BLK, 8), lambda i: (i, 0)),
            pl.BlockSpec((_BLK, 1), lambda i: (i, 0)),
            pl.BlockSpec((65, 64), lambda i: (0, 0)),
            pl.BlockSpec((1, 64), lambda i: (0, 0)),
        ],
        out_specs=pl.BlockSpec((_BLK, 64), lambda i: (i, 0)),
        out_shape=jax.ShapeDtypeStruct((n, 64), jnp.float32),
    )(sums, cnt, clause_x, w, b.reshape(1, 64))


def _tc_mean_relu(sums, cnt):
    n = sums.shape[0]

    def body(s_ref, c_ref, o_ref):
        cnt_col = jnp.maximum(c_ref[...][:, 0:1], 1.0)
        o_ref[...] = jnp.maximum(s_ref[...] / cnt_col, 0.0)

    return pl.pallas_call(
        body,
        grid=(n // _BLK,),
        in_specs=[
            pl.BlockSpec((_BLK, 64), lambda i: (i, 0)),
            pl.BlockSpec((_BLK, 8), lambda i: (i, 0)),
        ],
        out_specs=pl.BlockSpec((_BLK, 64), lambda i: (i, 0)),
        out_shape=jax.ShapeDtypeStruct((n, 64), jnp.float32),
    )(sums, cnt)


def _to_sliced(x):
    n = x.shape[0]
    return x.reshape(n, 8, 8).transpose(1, 0, 2).reshape(8 * n, 8)


def _from_sliced(y, n):
    return y.reshape(8, n, 8).transpose(1, 0, 2).reshape(n, 64)


def _tie(embs):
    n, v = embs.shape[0] // 2, embs.shape[1]
    y = embs.reshape(n, 2, v)
    pos, neg = y[:, 0, :], y[:, 1, :]
    cp = jnp.concatenate([pos, neg], axis=1)
    cn = jnp.concatenate([neg, pos], axis=1)
    return jnp.stack((cp, cn), axis=1).reshape(2 * n, 2 * v)


def kernel(lit_feat, clause_feat, edge_lit, edge_clause,
           W_l2c_0, b_l2c_0, W_c2l_0, b_c2l_0,
           W_l2c_1, b_l2c_1, W_c2l_1, b_c2l_1):
    el = jnp.asarray(edge_lit, jnp.int32)
    ec = jnp.asarray(edge_clause, jnp.int32)
    offs_l = (jnp.arange(8, dtype=jnp.int32) * N_LIT)[:, None]
    offs_c = (jnp.arange(8, dtype=jnp.int32) * N_CLAUSE)[:, None]
    gidx_l = (el[None, :] + offs_l).reshape(-1)
    gidx_c = (ec[None, :] + offs_c).reshape(-1)
    zeros = jnp.zeros((_RC, 8), jnp.float32)
    ones = jnp.ones((_KE, 8), jnp.float32)

    seg_c = _segsum_sc(N_CLAUSE)
    seg_l = _segsum_sc(N_LIT)
    cnt_c = _segcount_sc(N_CLAUSE)(ec, ones, zeros)
    cnt_l = _segcount_sc(N_LIT)(el, ones, zeros)

    def layer(lit_x, wl, bl, wc, bc):
        wh = _tc_linear(lit_x, wl, bl)
        s_c = seg_c(_to_sliced(wh), gidx_l, ec, zeros)
        wh2 = _tc_c2l(_from_sliced(s_c, N_CLAUSE), cnt_c, clause_feat, wc, bc)
        s_l = seg_l(_to_sliced(wh2), gidx_c, el, zeros)
        return _tc_mean_relu(_from_sliced(s_l, N_LIT), cnt_l)

    embs = _tie(layer(lit_feat, W_l2c_0, b_l2c_0, W_c2l_0, b_c2l_0))
    pre = layer(embs, W_l2c_1, b_l2c_1, W_c2l_1, b_c2l_1)
    return _tie(pre)


# speedup vs baseline: 4.7604x; 1.3321x over previous
import functools

import jax
import jax.numpy as jnp
from jax import lax
from jax.experimental import pallas as pl
from jax.experimental.pallas import tpu as pltpu
from jax.experimental.pallas import tpu_sc as plsc

N_LIT = 50000
N_CLAUSE = 150000
N_EDGES = 800000

_NC = 2
_NS = 16
_KE = 1000
_KEG = 2000
_EPC = N_EDGES // _NS
_NCH = _EPC // _KE
_NCHG = _EPC // _KEG
_RC = 1000


def _segsum_sc(nseg):
    nrch = nseg // _RC
    nz = -(-nrch // _NS)
    mesh = plsc.VectorSubcoreMesh(core_axis_name="c", subcore_axis_name="s")

    @functools.partial(
        pl.kernel, mesh=mesh,
        compiler_params=pltpu.CompilerParams(use_tc_tiling_on_sc=False),
        out_type=jax.ShapeDtypeStruct((8 * nseg, 8), jnp.float32),
        scratch_types=[
            pltpu.VMEM((_KEG,), jnp.int32),
            pltpu.VMEM((_KEG,), jnp.int32),
            pltpu.VMEM((_KEG,), jnp.int32),
            pltpu.VMEM((_KEG,), jnp.int32),
            pltpu.VMEM((_KEG, 8), jnp.float32),
            pltpu.VMEM((_KEG, 8), jnp.float32),
            pltpu.VMEM((_RC, 8), jnp.float32),
            pltpu.VMEM_SHARED((nseg, 8), jnp.float32),
            pltpu.SemaphoreType.DMA,
            pltpu.SemaphoreType.DMA,
        ],
    )
    def k(tbl, gidx, sidx, zeros, out,
          gv0, gv1, sv0, sv1, mv0, mv1, stg, acc, sem0, sem1):
        cid = lax.axis_index("c")
        sid = lax.axis_index("s")
        gvs, svs, mvs, sems = (gv0, gv1), (sv0, sv1), (mv0, mv1), (sem0, sem1)
        for p in range(4):
            sl = cid * 4 + p

            def zbody(z, _):
                j = sid + z * _NS

                @pl.when(j < nrch)
                def _():
                    pltpu.sync_copy(zeros, acc.at[pl.ds(j * _RC, _RC)])
                return _
            lax.fori_loop(0, nz, zbody, None)
            plsc.subcore_barrier()

            def issue(i, b):
                base = sid * _EPC + i * _KEG
                pltpu.sync_copy(
                    gidx.at[pl.ds(sl * N_EDGES + base, _KEG)], gvs[b])
                pltpu.sync_copy(sidx.at[pl.ds(base, _KEG)], svs[b])
                pltpu.async_copy(tbl.at[gvs[b]], mvs[b], sems[b])

            issue(0, 0)

            def ebody(i2, _):
                for b in range(2):
                    i = i2 * 2 + b

                    @pl.when(i + 1 < _NCHG)
                    def _():
                        issue(i + 1, (b + 1) % 2)
                    pltpu.make_async_copy(
                        tbl.at[gvs[b]], mvs[b], sems[b]).wait()
                    pltpu.sync_copy(mvs[b], acc.at[svs[b]], add=True)
                return _
            lax.fori_loop(0, _NCHG // 2, ebody, None)
            if _NCHG % 2:
                b = (_NCHG - 1) % 2
                pltpu.make_async_copy(tbl.at[gvs[b]], mvs[b], sems[b]).wait()
                pltpu.sync_copy(mvs[b], acc.at[svs[b]], add=True)
            plsc.subcore_barrier()

            def dbody(z, _):
                j = sid + z * _NS

                @pl.when(j < nrch)
                def _():
                    pltpu.sync_copy(acc.at[pl.ds(j * _RC, _RC)], stg)
                    pltpu.sync_copy(
                        stg, out.at[pl.ds(sl * nseg + j * _RC, _RC)])
                return _
            lax.fori_loop(0, nz, dbody, None)
            plsc.subcore_barrier()

    return k


def _segcount_sc(nseg):
    nrch = nseg // _RC
    nz = -(-nrch // _NS)
    mesh = plsc.VectorSubcoreMesh(core_axis_name="c", subcore_axis_name="s")

    @functools.partial(
        pl.kernel, mesh=mesh,
        compiler_params=pltpu.CompilerParams(use_tc_tiling_on_sc=False),
        out_type=jax.ShapeDtypeStruct((nseg, 8), jnp.float32),
        scratch_types=[
            pltpu.VMEM((_KE,), jnp.int32),
            pltpu.VMEM((_KE, 8), jnp.float32),
            pltpu.VMEM((_RC, 8), jnp.float32),
            pltpu.VMEM_SHARED((nseg, 8), jnp.float32),
        ],
    )
    def k(sidx, ones, zeros, out, sv, mv, stg, acc):
        cid = lax.axis_index("c")
        sid = lax.axis_index("s")

        @pl.when(cid == 0)
        def _():
            pltpu.sync_copy(ones, mv)

            def zbody(z, _):
                j = sid + z * _NS

                @pl.when(j < nrch)
                def _():
                    pltpu.sync_copy(zeros, acc.at[pl.ds(j * _RC, _RC)])
                return _
            lax.fori_loop(0, nz, zbody, None)
            plsc.subcore_barrier()

            def ebody(i, _):
                base = sid * _EPC + i * _KE
                pltpu.sync_copy(sidx.at[pl.ds(base, _KE)], sv)
                pltpu.sync_copy(mv, acc.at[sv], add=True)
                return _
            lax.fori_loop(0, _NCH, ebody, None)
            plsc.subcore_barrier()

            def dbody(z, _):
                j = sid + z * _NS

                @pl.when(j < nrch)
                def _():
                    pltpu.sync_copy(acc.at[pl.ds(j * _RC, _RC)], stg)
                    pltpu.sync_copy(stg, out.at[pl.ds(j * _RC, _RC)])
                return _
            lax.fori_loop(0, nz, dbody, None)

    return k


_BLK = 1000


def _tc_linear(x, w, b):
    n, kdim = x.shape

    def body(x_ref, w_ref, b_ref, o_ref):
        o_ref[...] = (
            jnp.dot(x_ref[...], w_ref[...],
                    preferred_element_type=jnp.float32) + b_ref[...])

    return pl.pallas_call(
        body,
        grid=(n // _BLK,),
        in_specs=[
            pl.BlockSpec((_BLK, kdim), lambda i: (i, 0)),
            pl.BlockSpec((kdim, 64), lambda i: (0, 0)),
            pl.BlockSpec((1, 64), lambda i: (0, 0)),
        ],
        out_specs=pl.BlockSpec((_BLK, 64), lambda i: (i, 0)),
        out_shape=jax.ShapeDtypeStruct((n, 64), jnp.float32),
    )(x, w, b.reshape(1, 64))


def _tc_c2l(sums, cnt, clause_x, w, b):
    n = cnt.shape[0]
    sums = sums.reshape(8, n, 8)

    def body(s_ref, c_ref, z_ref, w_ref, b_ref, o_ref):
        s = s_ref[...].transpose(1, 0, 2).reshape(_BLK, 64)
        cnt_col = jnp.maximum(c_ref[...][:, 0:1], 1.0)
        h = jnp.maximum(s / cnt_col, 0.0)
        o_ref[...] = (
            jnp.dot(h, w_ref[0:64, :], preferred_element_type=jnp.float32)
            + z_ref[...] * w_ref[64:65, :] + b_ref[...])

    return pl.pallas_call(
        body,
        grid=(n // _BLK,),
        in_specs=[
            pl.BlockSpec((8, _BLK, 8), lambda i: (0, i, 0)),
            pl.BlockSpec((_BLK, 8), lambda i: (i, 0)),
            pl.BlockSpec((_BLK, 1), lambda i: (i, 0)),
            pl.BlockSpec((65, 64), lambda i: (0, 0)),
            pl.BlockSpec((1, 64), lambda i: (0, 0)),
        ],
        out_specs=pl.BlockSpec((_BLK, 64), lambda i: (i, 0)),
        out_shape=jax.ShapeDtypeStruct((n, 64), jnp.float32),
    )(sums, cnt, clause_x, w, b.reshape(1, 64))


def _tc_mean_relu(sums, cnt):
    n = cnt.shape[0]
    sums = sums.reshape(8, n, 8)

    def body(s_ref, c_ref, o_ref):
        s = s_ref[...].transpose(1, 0, 2).reshape(_BLK, 64)
        cnt_col = jnp.maximum(c_ref[...][:, 0:1], 1.0)
        o_ref[...] = jnp.maximum(s / cnt_col, 0.0)

    return pl.pallas_call(
        body,
        grid=(n // _BLK,),
        in_specs=[
            pl.BlockSpec((8, _BLK, 8), lambda i: (0, i, 0)),
            pl.BlockSpec((_BLK, 8), lambda i: (i, 0)),
        ],
        out_specs=pl.BlockSpec((_BLK, 64), lambda i: (i, 0)),
        out_shape=jax.ShapeDtypeStruct((n, 64), jnp.float32),
    )(sums, cnt)


def _to_sliced(x):
    n = x.shape[0]
    return x.reshape(8 * n, 8)


def _tie(embs):
    n, v = embs.shape[0] // 2, embs.shape[1]
    y = embs.reshape(n, 2, v)
    pos, neg = y[:, 0, :], y[:, 1, :]
    cp = jnp.concatenate([pos, neg], axis=1)
    cn = jnp.concatenate([neg, pos], axis=1)
    return jnp.stack((cp, cn), axis=1).reshape(2 * n, 2 * v)


def kernel(lit_feat, clause_feat, edge_lit, edge_clause,
           W_l2c_0, b_l2c_0, W_c2l_0, b_c2l_0,
           W_l2c_1, b_l2c_1, W_c2l_1, b_c2l_1):
    el = jnp.asarray(edge_lit, jnp.int32)
    ec = jnp.asarray(edge_clause, jnp.int32)
    offs = jnp.arange(8, dtype=jnp.int32)[:, None]
    gidx_l = (8 * el[None, :] + offs).reshape(-1)
    gidx_c = (8 * ec[None, :] + offs).reshape(-1)
    zeros = jnp.zeros((_RC, 8), jnp.float32)
    ones = jnp.ones((_KE, 8), jnp.float32)

    seg_c = _segsum_sc(N_CLAUSE)
    seg_l = _segsum_sc(N_LIT)
    cnt_c = _segcount_sc(N_CLAUSE)(ec, ones, zeros)
    cnt_l = _segcount_sc(N_LIT)(el, ones, zeros)

    def layer(lit_x, wl, bl, wc, bc):
        wh = _tc_linear(lit_x, wl, bl)
        s_c = seg_c(_to_sliced(wh), gidx_l, ec, zeros)
        wh2 = _tc_c2l(s_c, cnt_c, clause_feat, wc, bc)
        s_l = seg_l(_to_sliced(wh2), gidx_c, el, zeros)
        return _tc_mean_relu(s_l, cnt_l)

    embs = _tie(layer(lit_feat, W_l2c_0, b_l2c_0, W_c2l_0, b_c2l_0))
    pre = layer(embs, W_l2c_1, b_l2c_1, W_c2l_1, b_c2l_1)
    return _tie(pre)
